# baseline (speedup 1.0000x reference)
import functools

import jax
import jax.numpy as jnp
from jax import lax
from jax.experimental import pallas as pl
from jax.experimental.pallas import tpu as pltpu
from jax.experimental.pallas import tpu_sc as plsc

NC = 2
NS = 16
NW = NC * NS

G = 128


def _sc_aggregate(h, src3, dst3, zeros_nd):
  N, D = h.shape
  NP = zeros_nd.shape[0]
  NCH = src3.shape[1]
  CH = src3.shape[2]
  RPT = NP // NS

  mesh = plsc.VectorSubcoreMesh(core_axis_name="c", subcore_axis_name="s")

  NBUF = 3
  scratch = (
      [pltpu.VMEM((CH,), jnp.int32) for _ in range(NBUF)] +
      [pltpu.VMEM((CH,), jnp.int32) for _ in range(NBUF)] +
      [pltpu.VMEM((CH, D), jnp.float32) for _ in range(NBUF)] +
      [pltpu.VMEM_SHARED((NP, D), jnp.float32)] +
      [pltpu.SemaphoreType.DMA for _ in range(NBUF)]
  )

  @functools.partial(
      pl.kernel,
      mesh=mesh,
      out_type=jax.ShapeDtypeStruct((NC, NP, D), jnp.float32),
      scratch_types=scratch,
  )
  def agg_kernel(h_hbm, src_hbm, dst_hbm, z_hbm, out_hbm, *sc):
    sidxs, didxs, rowss = sc[0:NBUF], sc[NBUF:2 * NBUF], sc[2 * NBUF:3 * NBUF]
    acc_sh = sc[3 * NBUF]
    sems = sc[3 * NBUF + 1:]
    bufs = list(zip(sidxs, didxs, rowss, sems))
    cid = lax.axis_index("c")
    sid = lax.axis_index("s")
    wid = sid * NC + cid

    pltpu.sync_copy(z_hbm.at[pl.ds(sid * RPT, RPT)],
                    acc_sh.at[pl.ds(sid * RPT, RPT)])
    plsc.subcore_barrier()

    for k, (sx, dx, rw, sm) in enumerate(bufs):
      pltpu.sync_copy(src_hbm.at[wid, k], sx)
      pltpu.async_copy(h_hbm.at[sx], rw, sm)
      pltpu.sync_copy(dst_hbm.at[wid, k], dx)

    def body(i, carry):
      for k, (sx, dx, rw, sm) in enumerate(bufs):
        g = NBUF * i + k
        pltpu.make_async_copy(h_hbm.at[sx], rw, sm).wait()
        pltpu.sync_copy(rw, acc_sh.at[dx], add=True)

        @pl.when(g + NBUF < NCH)
        def _(sx=sx, dx=dx, rw=rw, sm=sm, g=g):
          pltpu.sync_copy(src_hbm.at[wid, g + NBUF], sx)
          pltpu.async_copy(h_hbm.at[sx], rw, sm)
          pltpu.sync_copy(dst_hbm.at[wid, g + NBUF], dx)

      return carry

    lax.fori_loop(0, NCH // NBUF, body, 0)

    plsc.subcore_barrier()
    pltpu.sync_copy(acc_sh.at[pl.ds(sid * RPT, RPT)],
                    out_hbm.at[cid, pl.ds(sid * RPT, RPT)])

  return agg_kernel(h, src3, dst3, zeros_nd)


def _tc_layer(h, parts, batch3, W1, b1r, W2, b2r, eps11, B):
  N, D = h.shape
  H = W1.shape[1]
  NB = N // B

  def body(eps_ref, x_ref, a0_ref, a1_ref, b_ref, w1_ref, b1_ref,
           w2_ref, b2_ref, nodes_ref, pool_ref, cnt_ref):
    i = pl.program_id(0)
    e = eps_ref[0, 0]
    hin = (1.0 + e) * x_ref[...] + a0_ref[0] + a1_ref[0]
    t = jnp.dot(hin, w1_ref[...], preferred_element_type=jnp.float32)
    t = jnp.maximum(t + b1_ref[...], 0.0)
    out = jnp.dot(t, w2_ref[...], preferred_element_type=jnp.float32)
    out = jnp.maximum(out + b2_ref[...], 0.0)
    nodes_ref[...] = out

    bids = b_ref[0, 0, :]
    gids = lax.broadcasted_iota(jnp.int32, (G, B), 0)
    mask = (bids[None, :] == gids).astype(jnp.float32)

    @pl.when(i == 0)
    def _():
      pool_ref[...] = jnp.zeros_like(pool_ref)
      cnt_ref[...] = jnp.zeros_like(cnt_ref)

    pool_ref[...] += jnp.dot(mask, out, preferred_element_type=jnp.float32)
    cnt_ref[...] += jnp.sum(mask, axis=1, keepdims=True)

    @pl.when(i == NB - 1)
    def _():
      pool_ref[...] = pool_ref[...] / jnp.maximum(cnt_ref[...], 1.0)

  nodes, pool = pl.pallas_call(
      body,
      grid=(NB,),
      in_specs=[
          pl.BlockSpec(memory_space=pltpu.SMEM),
          pl.BlockSpec((B, D), lambda i: (i, 0)),
          pl.BlockSpec((1, B, D), lambda i: (0, i, 0)),
          pl.BlockSpec((1, B, D), lambda i: (1, i, 0)),
          pl.BlockSpec((1, 1, B), lambda i: (i, 0, 0)),
          pl.BlockSpec((D, H), lambda i: (0, 0)),
          pl.BlockSpec((1, H), lambda i: (0, 0)),
          pl.BlockSpec((H, H), lambda i: (0, 0)),
          pl.BlockSpec((1, H), lambda i: (0, 0)),
      ],
      out_specs=[
          pl.BlockSpec((B, H), lambda i: (i, 0)),
          pl.BlockSpec((G, H), lambda i: (0, 0)),
      ],
      out_shape=[
          jax.ShapeDtypeStruct((N, H), jnp.float32),
          jax.ShapeDtypeStruct((G, H), jnp.float32),
      ],
      scratch_shapes=[pltpu.VMEM((G, 1), jnp.float32)],
  )(eps11, h, parts, parts, batch3, W1, b1r, W2, b2r)
  return nodes, pool


def kernel(x, edge_index, batch,
           W1_0, b1_0, W2_0, b2_0, eps_0,
           W1_1, b1_1, W2_1, b2_1, eps_1,
           W1_2, b1_2, W2_2, b2_2, eps_2):
  N, D = x.shape
  src = edge_index[0].astype(jnp.int32)
  dst = edge_index[1].astype(jnp.int32)
  B = 1000
  batch3 = batch.astype(jnp.int32).reshape(N // B, 1, B)
  NP = ((N + 8 * NS - 1) // (8 * NS)) * (8 * NS)

  CH = 120
  E = src.shape[0]
  EPC = 3 * NW * CH
  EP = ((E + EPC - 1) // EPC) * EPC
  if EP > E and NP == N:
    NP += 8 * NS
  zeros_nd = jnp.zeros((NP, D), jnp.float32)
  if EP > E:
    pad_ar = jnp.arange(EP - E, dtype=jnp.int32)
    src = jnp.concatenate([src, pad_ar % N])
    dst = jnp.concatenate([dst, N + pad_ar % (NP - N)])
  NCH = EP // (NW * CH)
  src3 = src.reshape(NW, NCH, CH)
  dst3 = dst.reshape(NW, NCH, CH)

  params = [
      (W1_0, b1_0.reshape(1, -1), W2_0, b2_0.reshape(1, -1),
       eps_0.reshape(1, 1)),
      (W1_1, b1_1.reshape(1, -1), W2_1, b2_1.reshape(1, -1),
       eps_1.reshape(1, 1)),
      (W1_2, b1_2.reshape(1, -1), W2_2, b2_2.reshape(1, -1),
       eps_2.reshape(1, 1)),
  ]

  h = x
  nodes_list = []
  pool_list = []
  for (W1, b1r, W2, b2r, eps11) in params:
    parts = _sc_aggregate(h, src3, dst3, zeros_nd)
    h, pool = _tc_layer(h, parts, batch3, W1, b1r, W2, b2r, eps11, B)
    nodes_list.append(h)
    pool_list.append(pool)

  out_pool = jnp.concatenate(pool_list, axis=1)
  out_nodes = jnp.concatenate(nodes_list, axis=1)
  return (out_pool, out_nodes)


# speedup vs baseline: 1.3213x; 1.3213x over previous
import functools

import jax
import jax.numpy as jnp
from jax import lax
from jax.experimental import pallas as pl
from jax.experimental.pallas import tpu as pltpu
from jax.experimental.pallas import tpu_sc as plsc

NC = 2
NS = 16
NW = NC * NS

G = 128


def _sc_aggregate(h, pk3, zeros_nd):
  N, D = h.shape
  NP = zeros_nd.shape[0]
  NCH = pk3.shape[1]
  CH = pk3.shape[2]
  RPT = NP // NS

  mesh = plsc.VectorSubcoreMesh(core_axis_name="c", subcore_axis_name="s")

  scratch = [
      pltpu.VMEM((NCH, CH), jnp.int32),
      pltpu.VMEM((CH,), jnp.int32),
      pltpu.VMEM((CH,), jnp.int32),
      pltpu.VMEM((CH,), jnp.int32),
      pltpu.VMEM((CH, D), jnp.float32),
      pltpu.VMEM((CH, D), jnp.float32),
      pltpu.VMEM_SHARED((NP, D), jnp.float32),
      pltpu.SemaphoreType.DMA,
      pltpu.SemaphoreType.DMA,
  ]

  NV = CH // 16

  @functools.partial(
      pl.kernel,
      mesh=mesh,
      out_type=jax.ShapeDtypeStruct((NC, NP, D), jnp.float32),
      scratch_types=scratch,
  )
  def agg_kernel(h_hbm, pk_hbm, z_hbm, out_hbm,
                 pk, sidx_a, sidx_b, didx, rows_a, rows_b, acc_sh,
                 sem_a, sem_b):
    cid = lax.axis_index("c")
    sid = lax.axis_index("s")
    wid = sid * NC + cid

    def unpack_src(g, dst_ref):
      for j in range(NV):
        rec = pk[g, pl.ds(16 * j, 16)]
        dst_ref[pl.ds(16 * j, 16)] = rec & 0x3FFF

    def unpack_dst(g, dst_ref):
      for j in range(NV):
        rec = pk[g, pl.ds(16 * j, 16)]
        dst_ref[pl.ds(16 * j, 16)] = lax.shift_right_logical(rec, 14)

    pltpu.sync_copy(pk_hbm.at[wid], pk)
    pltpu.sync_copy(z_hbm.at[pl.ds(sid * RPT, RPT)],
                    acc_sh.at[pl.ds(sid * RPT, RPT)])
    plsc.subcore_barrier()

    unpack_src(0, sidx_a)
    pltpu.async_copy(h_hbm.at[sidx_a], rows_a, sem_a)
    unpack_src(1, sidx_b)
    pltpu.async_copy(h_hbm.at[sidx_b], rows_b, sem_b)

    def body(i, carry):
      g = 2 * i
      pltpu.make_async_copy(h_hbm.at[sidx_a], rows_a, sem_a).wait()
      unpack_dst(g, didx)
      pltpu.sync_copy(rows_a, acc_sh.at[didx], add=True)

      @pl.when(g + 2 < NCH)
      def _():
        unpack_src(g + 2, sidx_a)
        pltpu.async_copy(h_hbm.at[sidx_a], rows_a, sem_a)

      pltpu.make_async_copy(h_hbm.at[sidx_b], rows_b, sem_b).wait()
      unpack_dst(g + 1, didx)
      pltpu.sync_copy(rows_b, acc_sh.at[didx], add=True)

      @pl.when(g + 3 < NCH)
      def _():
        unpack_src(g + 3, sidx_b)
        pltpu.async_copy(h_hbm.at[sidx_b], rows_b, sem_b)

      return carry

    lax.fori_loop(0, NCH // 2, body, 0)

    plsc.subcore_barrier()
    pltpu.sync_copy(acc_sh.at[pl.ds(sid * RPT, RPT)],
                    out_hbm.at[cid, pl.ds(sid * RPT, RPT)])

  return agg_kernel(h, pk3, zeros_nd)


def _tc_layer(h, parts, batch3, W1, b1r, W2, b2r, eps11, B):
  N, D = h.shape
  H = W1.shape[1]
  NB = N // B

  def body(eps_ref, x_ref, a0_ref, a1_ref, b_ref, w1_ref, b1_ref,
           w2_ref, b2_ref, nodes_ref, pool_ref, cnt_ref):
    i = pl.program_id(0)
    e = eps_ref[0, 0]
    hin = (1.0 + e) * x_ref[...] + a0_ref[0] + a1_ref[0]
    t = jnp.dot(hin, w1_ref[...], preferred_element_type=jnp.float32)
    t = jnp.maximum(t + b1_ref[...], 0.0)
    out = jnp.dot(t, w2_ref[...], preferred_element_type=jnp.float32)
    out = jnp.maximum(out + b2_ref[...], 0.0)
    nodes_ref[...] = out

    bids = b_ref[0, 0, :]
    gids = lax.broadcasted_iota(jnp.int32, (G, B), 0)
    mask = (bids[None, :] == gids).astype(jnp.float32)

    @pl.when(i == 0)
    def _():
      pool_ref[...] = jnp.zeros_like(pool_ref)
      cnt_ref[...] = jnp.zeros_like(cnt_ref)

    pool_ref[...] += jnp.dot(mask, out, preferred_element_type=jnp.float32)
    cnt_ref[...] += jnp.sum(mask, axis=1, keepdims=True)

    @pl.when(i == NB - 1)
    def _():
      pool_ref[...] = pool_ref[...] / jnp.maximum(cnt_ref[...], 1.0)

  nodes, pool = pl.pallas_call(
      body,
      grid=(NB,),
      in_specs=[
          pl.BlockSpec(memory_space=pltpu.SMEM),
          pl.BlockSpec((B, D), lambda i: (i, 0)),
          pl.BlockSpec((1, B, D), lambda i: (0, i, 0)),
          pl.BlockSpec((1, B, D), lambda i: (1, i, 0)),
          pl.BlockSpec((1, 1, B), lambda i: (i, 0, 0)),
          pl.BlockSpec((D, H), lambda i: (0, 0)),
          pl.BlockSpec((1, H), lambda i: (0, 0)),
          pl.BlockSpec((H, H), lambda i: (0, 0)),
          pl.BlockSpec((1, H), lambda i: (0, 0)),
      ],
      out_specs=[
          pl.BlockSpec((B, H), lambda i: (i, 0)),
          pl.BlockSpec((G, H), lambda i: (0, 0)),
      ],
      out_shape=[
          jax.ShapeDtypeStruct((N, H), jnp.float32),
          jax.ShapeDtypeStruct((G, H), jnp.float32),
      ],
      scratch_shapes=[pltpu.VMEM((G, 1), jnp.float32)],
  )(eps11, h, parts, parts, batch3, W1, b1r, W2, b2r)
  return nodes, pool


def kernel(x, edge_index, batch,
           W1_0, b1_0, W2_0, b2_0, eps_0,
           W1_1, b1_1, W2_1, b2_1, eps_1,
           W1_2, b1_2, W2_2, b2_2, eps_2):
  N, D = x.shape
  src = edge_index[0].astype(jnp.int32)
  dst = edge_index[1].astype(jnp.int32)
  B = 1000
  batch3 = batch.astype(jnp.int32).reshape(N // B, 1, B)
  NP = ((N + 8 * NS - 1) // (8 * NS)) * (8 * NS)

  CH = 128
  E = src.shape[0]
  EPC = 2 * NW * CH
  EP = ((E + EPC - 1) // EPC) * EPC
  if EP > E and NP == N:
    NP += 8 * NS
  zeros_nd = jnp.zeros((NP, D), jnp.float32)
  if EP > E:
    pad_ar = jnp.arange(EP - E, dtype=jnp.int32)
    src = jnp.concatenate([src, pad_ar % N])
    dst = jnp.concatenate([dst, N + pad_ar % (NP - N)])
  NCH = EP // (NW * CH)
  pk3 = (src | (dst << 14)).reshape(NW, NCH, CH)

  params = [
      (W1_0, b1_0.reshape(1, -1), W2_0, b2_0.reshape(1, -1),
       eps_0.reshape(1, 1)),
      (W1_1, b1_1.reshape(1, -1), W2_1, b2_1.reshape(1, -1),
       eps_1.reshape(1, 1)),
      (W1_2, b1_2.reshape(1, -1), W2_2, b2_2.reshape(1, -1),
       eps_2.reshape(1, 1)),
  ]

  h = x
  nodes_list = []
  pool_list = []
  for (W1, b1r, W2, b2r, eps11) in params:
    parts = _sc_aggregate(h, pk3, zeros_nd)
    h, pool = _tc_layer(h, parts, batch3, W1, b1r, W2, b2r, eps11, B)
    nodes_list.append(h)
    pool_list.append(pool)

  out_pool = jnp.concatenate(pool_list, axis=1)
  out_nodes = jnp.concatenate(nodes_list, axis=1)
  return (out_pool, out_nodes)


# speedup vs baseline: 1.3383x; 1.0129x over previous
import functools

import jax
import jax.numpy as jnp
from jax import lax
from jax.experimental import pallas as pl
from jax.experimental.pallas import tpu as pltpu
from jax.experimental.pallas import tpu_sc as plsc

NC = 2
NS = 16
NW = NC * NS

G = 128


def _sc_aggregate(h, pk3, zeros_nd):
  N, D = h.shape
  NP = zeros_nd.shape[0]
  NCH = pk3.shape[1]
  CH = pk3.shape[2]
  RPT = NP // NS

  mesh = plsc.VectorSubcoreMesh(core_axis_name="c", subcore_axis_name="s")

  scratch = [
      pltpu.VMEM((NCH, CH), jnp.int32),
      pltpu.VMEM((CH,), jnp.int32),
      pltpu.VMEM((CH,), jnp.int32),
      pltpu.VMEM((CH,), jnp.int32),
      pltpu.VMEM((CH, D), jnp.float32),
      pltpu.VMEM((CH, D), jnp.float32),
      pltpu.VMEM_SHARED((NP, D), jnp.float32),
      pltpu.SemaphoreType.DMA,
      pltpu.SemaphoreType.DMA,
  ]

  NV = CH // 16

  @functools.partial(
      pl.kernel,
      mesh=mesh,
      out_type=jax.ShapeDtypeStruct((NC, NP, D), jnp.float32),
      scratch_types=scratch,
  )
  def agg_kernel(h_hbm, pk_hbm, z_hbm, out_hbm,
                 pk, sidx_a, sidx_b, didx, rows_a, rows_b, acc_sh,
                 sem_a, sem_b):
    cid = lax.axis_index("c")
    sid = lax.axis_index("s")
    wid = sid * NC + cid

    def unpack_src(g, dst_ref):
      for j in range(NV):
        rec = pk[g, pl.ds(16 * j, 16)]
        dst_ref[pl.ds(16 * j, 16)] = rec & 0x3FFF

    def unpack_dst(g, dst_ref):
      for j in range(NV):
        rec = pk[g, pl.ds(16 * j, 16)]
        dst_ref[pl.ds(16 * j, 16)] = lax.shift_right_logical(rec, 14)

    pltpu.sync_copy(pk_hbm.at[wid], pk)
    pltpu.sync_copy(z_hbm.at[pl.ds(sid * RPT, RPT)],
                    acc_sh.at[pl.ds(sid * RPT, RPT)])
    plsc.subcore_barrier()

    unpack_src(0, sidx_a)
    pltpu.async_copy(h_hbm.at[sidx_a], rows_a, sem_a)
    unpack_src(1, sidx_b)
    pltpu.async_copy(h_hbm.at[sidx_b], rows_b, sem_b)

    def body(i, carry):
      g = 2 * i
      pltpu.make_async_copy(h_hbm.at[sidx_a], rows_a, sem_a).wait()
      unpack_dst(g, didx)
      pltpu.sync_copy(rows_a, acc_sh.at[didx], add=True)

      @pl.when(g + 2 < NCH)
      def _():
        unpack_src(g + 2, sidx_a)
        pltpu.async_copy(h_hbm.at[sidx_a], rows_a, sem_a)

      pltpu.make_async_copy(h_hbm.at[sidx_b], rows_b, sem_b).wait()
      unpack_dst(g + 1, didx)
      pltpu.sync_copy(rows_b, acc_sh.at[didx], add=True)

      @pl.when(g + 3 < NCH)
      def _():
        unpack_src(g + 3, sidx_b)
        pltpu.async_copy(h_hbm.at[sidx_b], rows_b, sem_b)

      return carry

    lax.fori_loop(0, NCH // 2, body, 0)

    plsc.subcore_barrier()
    pltpu.sync_copy(acc_sh.at[pl.ds(sid * RPT, RPT)],
                    out_hbm.at[cid, pl.ds(sid * RPT, RPT)])

  return agg_kernel(h, pk3, zeros_nd)


def _tc_layer(h, parts, batch3, W1, b1r, W2, b2r, eps11, B):
  N, D = h.shape
  H = W1.shape[1]
  NB = N // B

  def body(eps_ref, x_ref, a0_ref, a1_ref, b_ref, w1_ref, b1_ref,
           w2_ref, b2_ref, nodes_ref, pool_ref, cnt_ref):
    i = pl.program_id(0)
    e = eps_ref[0, 0]
    hin = (1.0 + e) * x_ref[...] + a0_ref[0] + a1_ref[0]
    t = jnp.dot(hin, w1_ref[...], preferred_element_type=jnp.float32)
    t = jnp.maximum(t + b1_ref[...], 0.0)
    out = jnp.dot(t, w2_ref[...], preferred_element_type=jnp.float32)
    out = jnp.maximum(out + b2_ref[...], 0.0)
    nodes_ref[...] = out

    bids = b_ref[0, 0, :]
    gids = lax.broadcasted_iota(jnp.int32, (G, B), 0)
    mask = (bids[None, :] == gids).astype(jnp.float32)

    @pl.when(i == 0)
    def _():
      pool_ref[...] = jnp.zeros_like(pool_ref)
      cnt_ref[...] = jnp.zeros_like(cnt_ref)

    pool_ref[...] += jnp.dot(mask, out, preferred_element_type=jnp.float32)
    cnt_ref[...] += jnp.sum(mask, axis=1, keepdims=True)

    @pl.when(i == NB - 1)
    def _():
      pool_ref[...] = pool_ref[...] / jnp.maximum(cnt_ref[...], 1.0)

  nodes, pool = pl.pallas_call(
      body,
      grid=(NB,),
      in_specs=[
          pl.BlockSpec(memory_space=pltpu.SMEM),
          pl.BlockSpec((B, D), lambda i: (i, 0)),
          pl.BlockSpec((1, B, D), lambda i: (0, i, 0)),
          pl.BlockSpec((1, B, D), lambda i: (1, i, 0)),
          pl.BlockSpec((1, 1, B), lambda i: (i, 0, 0)),
          pl.BlockSpec((D, H), lambda i: (0, 0)),
          pl.BlockSpec((1, H), lambda i: (0, 0)),
          pl.BlockSpec((H, H), lambda i: (0, 0)),
          pl.BlockSpec((1, H), lambda i: (0, 0)),
      ],
      out_specs=[
          pl.BlockSpec((B, H), lambda i: (i, 0)),
          pl.BlockSpec((G, H), lambda i: (0, 0)),
      ],
      out_shape=[
          jax.ShapeDtypeStruct((N, H), jnp.float32),
          jax.ShapeDtypeStruct((G, H), jnp.float32),
      ],
      scratch_shapes=[pltpu.VMEM((G, 1), jnp.float32)],
  )(eps11, h, parts, parts, batch3, W1, b1r, W2, b2r)
  return nodes, pool


def _tc_layer_last(h, parts, batch3, W1, b1r, W2, b2r, eps11, B,
                   h1, h2, p1, p2):
  N, D = h.shape
  H = W1.shape[1]
  NB = N // B

  def body(eps_ref, x_ref, a0_ref, a1_ref, b_ref, w1_ref, b1_ref,
           w2_ref, b2_ref, h1_ref, h2_ref, p1_ref, p2_ref,
           nodes_ref, pool_ref, pacc_ref, cnt_ref):
    i = pl.program_id(0)
    e = eps_ref[0, 0]
    hin = (1.0 + e) * x_ref[...] + a0_ref[0] + a1_ref[0]
    t = jnp.dot(hin, w1_ref[...], preferred_element_type=jnp.float32)
    t = jnp.maximum(t + b1_ref[...], 0.0)
    out = jnp.dot(t, w2_ref[...], preferred_element_type=jnp.float32)
    out = jnp.maximum(out + b2_ref[...], 0.0)
    nodes_ref[...] = jnp.concatenate([h1_ref[...], h2_ref[...], out], axis=1)

    bids = b_ref[0, 0, :]
    gids = lax.broadcasted_iota(jnp.int32, (G, B), 0)
    mask = (bids[None, :] == gids).astype(jnp.float32)

    @pl.when(i == 0)
    def _():
      pacc_ref[...] = jnp.zeros_like(pacc_ref)
      cnt_ref[...] = jnp.zeros_like(cnt_ref)

    pacc_ref[...] += jnp.dot(mask, out, preferred_element_type=jnp.float32)
    cnt_ref[...] += jnp.sum(mask, axis=1, keepdims=True)

    @pl.when(i == NB - 1)
    def _():
      p3 = pacc_ref[...] / jnp.maximum(cnt_ref[...], 1.0)
      pool_ref[...] = jnp.concatenate([p1_ref[...], p2_ref[...], p3], axis=1)

  nodes_all, pool_all = pl.pallas_call(
      body,
      grid=(NB,),
      in_specs=[
          pl.BlockSpec(memory_space=pltpu.SMEM),
          pl.BlockSpec((B, D), lambda i: (i, 0)),
          pl.BlockSpec((1, B, D), lambda i: (0, i, 0)),
          pl.BlockSpec((1, B, D), lambda i: (1, i, 0)),
          pl.BlockSpec((1, 1, B), lambda i: (i, 0, 0)),
          pl.BlockSpec((D, H), lambda i: (0, 0)),
          pl.BlockSpec((1, H), lambda i: (0, 0)),
          pl.BlockSpec((H, H), lambda i: (0, 0)),
          pl.BlockSpec((1, H), lambda i: (0, 0)),
          pl.BlockSpec((B, H), lambda i: (i, 0)),
          pl.BlockSpec((B, H), lambda i: (i, 0)),
          pl.BlockSpec((G, H), lambda i: (0, 0)),
          pl.BlockSpec((G, H), lambda i: (0, 0)),
      ],
      out_specs=[
          pl.BlockSpec((B, 3 * H), lambda i: (i, 0)),
          pl.BlockSpec((G, 3 * H), lambda i: (0, 0)),
      ],
      out_shape=[
          jax.ShapeDtypeStruct((N, 3 * H), jnp.float32),
          jax.ShapeDtypeStruct((G, 3 * H), jnp.float32),
      ],
      scratch_shapes=[
          pltpu.VMEM((G, H), jnp.float32),
          pltpu.VMEM((G, 1), jnp.float32),
      ],
  )(eps11, h, parts, parts, batch3, W1, b1r, W2, b2r, h1, h2, p1, p2)
  return nodes_all, pool_all


def kernel(x, edge_index, batch,
           W1_0, b1_0, W2_0, b2_0, eps_0,
           W1_1, b1_1, W2_1, b2_1, eps_1,
           W1_2, b1_2, W2_2, b2_2, eps_2):
  N, D = x.shape
  src = edge_index[0].astype(jnp.int32)
  dst = edge_index[1].astype(jnp.int32)
  B = 1000
  batch3 = batch.astype(jnp.int32).reshape(N // B, 1, B)
  NP = ((N + 8 * NS - 1) // (8 * NS)) * (8 * NS)

  CH = 128
  E = src.shape[0]
  EPC = 2 * NW * CH
  EP = ((E + EPC - 1) // EPC) * EPC
  if EP > E and NP == N:
    NP += 8 * NS
  zeros_nd = jnp.zeros((NP, D), jnp.float32)
  if EP > E:
    pad_ar = jnp.arange(EP - E, dtype=jnp.int32)
    src = jnp.concatenate([src, pad_ar % N])
    dst = jnp.concatenate([dst, N + pad_ar % (NP - N)])
  NCH = EP // (NW * CH)
  pk3 = (src | (dst << 14)).reshape(NW, NCH, CH)

  params = [
      (W1_0, b1_0.reshape(1, -1), W2_0, b2_0.reshape(1, -1),
       eps_0.reshape(1, 1)),
      (W1_1, b1_1.reshape(1, -1), W2_1, b2_1.reshape(1, -1),
       eps_1.reshape(1, 1)),
      (W1_2, b1_2.reshape(1, -1), W2_2, b2_2.reshape(1, -1),
       eps_2.reshape(1, 1)),
  ]

  h = x
  nodes_list = []
  pool_list = []
  for (W1, b1r, W2, b2r, eps11) in params[:2]:
    parts = _sc_aggregate(h, pk3, zeros_nd)
    h, pool = _tc_layer(h, parts, batch3, W1, b1r, W2, b2r, eps11, B)
    nodes_list.append(h)
    pool_list.append(pool)

  (W1, b1r, W2, b2r, eps11) = params[2]
  parts = _sc_aggregate(h, pk3, zeros_nd)
  out_nodes, out_pool = _tc_layer_last(
      h, parts, batch3, W1, b1r, W2, b2r, eps11, B,
      nodes_list[0], nodes_list[1], pool_list[0], pool_list[1])
  return (out_pool, out_nodes)
